# Initial kernel scaffold; baseline (speedup 1.0000x reference)
import functools

import jax
import jax.numpy as jnp
from jax import lax
from jax.experimental import pallas as pl
from jax.experimental.pallas import tpu as pltpu
from jax.experimental.pallas import tpu_sc as plsc

VOCAB = 100000
D_EMB = 128
MAX_SEQ = 2048
D_MODEL = 1024
LN_EPS = 1e-12

N_TOK = 8192
NW = 32
TOK_PER_TILE = N_TOK // NW
ROWS_PER_GATHER = 128

TC_BLOCK = 256


def _make_sc_gather():
  mesh = plsc.VectorSubcoreMesh(core_axis_name="c", subcore_axis_name="s")

  @functools.partial(
      pl.kernel,
      mesh=mesh,
      out_type=jax.ShapeDtypeStruct((N_TOK, D_EMB), jnp.float32),
      scratch_types=[
          pltpu.VMEM((2, ROWS_PER_GATHER), jnp.int32),
          pltpu.VMEM((TOK_PER_TILE, D_EMB), jnp.float32),
          pltpu.SemaphoreType.DMA,
      ],
  )
  def gather_kernel(ids_hbm, table_hbm, out_hbm, idx_v, rows_v, sem):
    c = lax.axis_index("c")
    s = lax.axis_index("s")
    wid = s * 2 + c
    pltpu.sync_copy(ids_hbm.at[pl.ds(wid * 2, 2)], idx_v)
    cp0 = pltpu.async_copy(
        table_hbm.at[idx_v.at[0]], rows_v.at[pl.ds(0, ROWS_PER_GATHER)], sem)
    cp1 = pltpu.async_copy(
        table_hbm.at[idx_v.at[1]],
        rows_v.at[pl.ds(ROWS_PER_GATHER, ROWS_PER_GATHER)], sem)
    cp0.wait()
    cp1.wait()
    pltpu.sync_copy(rows_v, out_hbm.at[pl.ds(wid * TOK_PER_TILE, TOK_PER_TILE)])

  return gather_kernel


_sc_gather = _make_sc_gather()


def _tc_body(gath_ref, pos_ref, tid_ref, temb_ref, scale_ref, bias_ref,
             dk_ref, db_ref, out_ref):
  x = gath_ref[...] + pos_ref[...]
  t = tid_ref[...].astype(jnp.float32)
  te0 = temb_ref[0:1, :]
  te1 = temb_ref[1:2, :]
  x = x + te0 + t * (te1 - te0)
  mean = jnp.mean(x, axis=1, keepdims=True)
  xc = x - mean
  var = jnp.mean(xc * xc, axis=1, keepdims=True)
  y = xc * lax.rsqrt(var + LN_EPS)
  y = y * scale_ref[...] + bias_ref[...]
  out_ref[...] = (
      jnp.dot(y, dk_ref[...], preferred_element_type=jnp.float32)
      + db_ref[...])


def kernel(input_ids, type_ids, word_emb, pos_emb, type_emb, ln_scale,
           ln_bias, dense_kernel, dense_bias):
  batch, seq = input_ids.shape
  n_tok = batch * seq

  ids2 = input_ids.reshape(NW * 2, ROWS_PER_GATHER).astype(jnp.int32)
  gathered = _sc_gather(ids2, word_emb)

  pos2 = pos_emb.reshape(MAX_SEQ, D_EMB)[:seq]
  tids = type_ids.reshape(n_tok, 1).astype(jnp.int32)
  scale2 = ln_scale.reshape(1, D_EMB)
  bias2 = ln_bias.reshape(1, D_EMB)
  db2 = dense_bias.reshape(1, D_MODEL)

  n_blocks = n_tok // TC_BLOCK
  pos_blocks = seq // TC_BLOCK

  out = pl.pallas_call(
      _tc_body,
      grid=(n_blocks,),
      in_specs=[
          pl.BlockSpec((TC_BLOCK, D_EMB), lambda i: (i, 0)),
          pl.BlockSpec((TC_BLOCK, D_EMB), lambda i: (i % pos_blocks, 0)),
          pl.BlockSpec((TC_BLOCK, 1), lambda i: (i, 0)),
          pl.BlockSpec((2, D_EMB), lambda i: (0, 0)),
          pl.BlockSpec((1, D_EMB), lambda i: (0, 0)),
          pl.BlockSpec((1, D_EMB), lambda i: (0, 0)),
          pl.BlockSpec((D_EMB, D_MODEL), lambda i: (0, 0)),
          pl.BlockSpec((1, D_MODEL), lambda i: (0, 0)),
      ],
      out_specs=pl.BlockSpec((TC_BLOCK, D_MODEL), lambda i: (i, 0)),
      out_shape=jax.ShapeDtypeStruct((n_tok, D_MODEL), jnp.float32),
  )(gathered, pos2, tids, type_emb, scale2, bias2, dense_kernel, db2)

  return out.reshape(batch, seq, D_MODEL)


# speedup vs baseline: 1.0763x; 1.0763x over previous
import functools

import jax
import jax.numpy as jnp
from jax import lax
from jax.experimental import pallas as pl
from jax.experimental.pallas import tpu as pltpu
from jax.experimental.pallas import tpu_sc as plsc

VOCAB = 100000
D_EMB = 128
MAX_SEQ = 2048
D_MODEL = 1024
LN_EPS = 1e-12

N_TOK = 8192
NW = 32
TOK_PER_TILE = N_TOK // NW
ROWS_PER_GATHER = 128

TC_BLOCK = 256


@functools.cache
def _make_sc_gather():
  mesh = plsc.VectorSubcoreMesh(core_axis_name="c", subcore_axis_name="s")

  @functools.partial(
      pl.kernel,
      mesh=mesh,
      out_type=jax.ShapeDtypeStruct((N_TOK, D_EMB), jnp.float32),
      scratch_types=[
          pltpu.VMEM((2, ROWS_PER_GATHER), jnp.int32),
          pltpu.VMEM((TOK_PER_TILE, D_EMB), jnp.float32),
          pltpu.SemaphoreType.DMA,
      ],
  )
  def gather_kernel(ids_hbm, table_hbm, out_hbm, idx_v, rows_v, sem):
    c = lax.axis_index("c")
    s = lax.axis_index("s")
    wid = s * 2 + c
    pltpu.sync_copy(ids_hbm.at[pl.ds(wid * 2, 2)], idx_v)
    cp0 = pltpu.async_copy(
        table_hbm.at[idx_v.at[0]], rows_v.at[pl.ds(0, ROWS_PER_GATHER)], sem)
    cp1 = pltpu.async_copy(
        table_hbm.at[idx_v.at[1]],
        rows_v.at[pl.ds(ROWS_PER_GATHER, ROWS_PER_GATHER)], sem)
    cp0.wait()
    cp1.wait()
    pltpu.sync_copy(rows_v, out_hbm.at[pl.ds(wid * TOK_PER_TILE, TOK_PER_TILE)])

  return gather_kernel


def _tc_body(gath_ref, pos_ref, tid_ref, temb_ref, scale_ref, bias_ref,
             dk_ref, db_ref, out_ref):
  x = gath_ref[...] + pos_ref[...]
  t = tid_ref[...].astype(jnp.float32)
  te0 = temb_ref[0:1, :]
  te1 = temb_ref[1:2, :]
  x = x + te0 + t * (te1 - te0)
  mean = jnp.mean(x, axis=1, keepdims=True)
  xc = x - mean
  var = jnp.mean(xc * xc, axis=1, keepdims=True)
  y = xc * lax.rsqrt(var + LN_EPS)
  y = y * scale_ref[...] + bias_ref[...]
  out_ref[...] = (
      jnp.dot(y, dk_ref[...], preferred_element_type=jnp.float32)
      + db_ref[...])


def kernel(input_ids, type_ids, word_emb, pos_emb, type_emb, ln_scale,
           ln_bias, dense_kernel, dense_bias):
  batch, seq = input_ids.shape
  n_tok = batch * seq

  ids2 = input_ids.reshape(NW * 2, ROWS_PER_GATHER).astype(jnp.int32)
  gathered = _make_sc_gather()(ids2, word_emb)

  pos2 = pos_emb.reshape(MAX_SEQ, D_EMB)[:seq]
  tids = type_ids.reshape(n_tok, 1).astype(jnp.int32)
  scale2 = ln_scale.reshape(1, D_EMB)
  bias2 = ln_bias.reshape(1, D_EMB)
  db2 = dense_bias.reshape(1, D_MODEL)

  n_blocks = n_tok // TC_BLOCK
  pos_blocks = seq // TC_BLOCK

  out = pl.pallas_call(
      _tc_body,
      grid=(n_blocks,),
      in_specs=[
          pl.BlockSpec((TC_BLOCK, D_EMB), lambda i: (i, 0)),
          pl.BlockSpec((TC_BLOCK, D_EMB), lambda i: (i % pos_blocks, 0)),
          pl.BlockSpec((TC_BLOCK, 1), lambda i: (i, 0)),
          pl.BlockSpec((2, D_EMB), lambda i: (0, 0)),
          pl.BlockSpec((1, D_EMB), lambda i: (0, 0)),
          pl.BlockSpec((1, D_EMB), lambda i: (0, 0)),
          pl.BlockSpec((D_EMB, D_MODEL), lambda i: (0, 0)),
          pl.BlockSpec((1, D_MODEL), lambda i: (0, 0)),
      ],
      out_specs=pl.BlockSpec((TC_BLOCK, D_MODEL), lambda i: (i, 0)),
      out_shape=jax.ShapeDtypeStruct((n_tok, D_MODEL), jnp.float32),
  )(gathered, pos2, tids, type_emb, scale2, bias2, dense_kernel, db2)

  return out.reshape(batch, seq, D_MODEL)


# speedup vs baseline: 1.2706x; 1.1805x over previous
import functools

import jax
import jax.numpy as jnp
from jax import lax
from jax.experimental import pallas as pl
from jax.experimental.pallas import tpu as pltpu
from jax.experimental.pallas import tpu_sc as plsc

VOCAB = 100000
D_EMB = 128
MAX_SEQ = 2048
D_MODEL = 1024
LN_EPS = 1e-12

N_TOK = 8192
NW = 32
TOK_PER_TILE = N_TOK // NW
ROWS_PER_GATHER = 128

TC_BLOCK = 512


@functools.cache
def _make_sc_gather():
  mesh = plsc.VectorSubcoreMesh(core_axis_name="c", subcore_axis_name="s")

  @functools.partial(
      pl.kernel,
      mesh=mesh,
      out_type=jax.ShapeDtypeStruct((N_TOK, D_EMB), jnp.float32),
      scratch_types=[
          pltpu.VMEM((2, ROWS_PER_GATHER), jnp.int32),
          pltpu.VMEM((TOK_PER_TILE, D_EMB), jnp.float32),
          pltpu.SemaphoreType.DMA,
      ],
  )
  def gather_kernel(ids_hbm, table_hbm, out_hbm, idx_v, rows_v, sem):
    c = lax.axis_index("c")
    s = lax.axis_index("s")
    wid = s * 2 + c
    pltpu.sync_copy(ids_hbm.at[pl.ds(wid * 2, 2)], idx_v)
    cp0 = pltpu.async_copy(
        table_hbm.at[idx_v.at[0]], rows_v.at[pl.ds(0, ROWS_PER_GATHER)], sem)
    cp1 = pltpu.async_copy(
        table_hbm.at[idx_v.at[1]],
        rows_v.at[pl.ds(ROWS_PER_GATHER, ROWS_PER_GATHER)], sem)
    cp0.wait()
    cp1.wait()
    pltpu.sync_copy(rows_v, out_hbm.at[pl.ds(wid * TOK_PER_TILE, TOK_PER_TILE)])

  return gather_kernel


def _tc_body(gath_ref, pos_ref, tid_ref, temb_ref, scale_ref, bias_ref,
             dk_ref, db_ref, out_ref):
  x = gath_ref[...] + pos_ref[...]
  t = tid_ref[...].astype(jnp.float32)
  te0 = temb_ref[0:1, :]
  te1 = temb_ref[1:2, :]
  x = x + te0 + t * (te1 - te0)
  mean = jnp.mean(x, axis=1, keepdims=True)
  xc = x - mean
  var = jnp.mean(xc * xc, axis=1, keepdims=True)
  y = xc * lax.rsqrt(var + LN_EPS)
  y = y * scale_ref[...] + bias_ref[...]
  out_ref[...] = (
      jnp.dot(y, dk_ref[...], preferred_element_type=jnp.float32)
      + db_ref[...])


def kernel(input_ids, type_ids, word_emb, pos_emb, type_emb, ln_scale,
           ln_bias, dense_kernel, dense_bias):
  batch, seq = input_ids.shape
  n_tok = batch * seq

  ids2 = input_ids.reshape(NW * 2, ROWS_PER_GATHER).astype(jnp.int32)
  gathered = _make_sc_gather()(ids2, word_emb)

  pos2 = pos_emb.reshape(MAX_SEQ, D_EMB)[:seq]
  tids = type_ids.reshape(n_tok, 1).astype(jnp.int32)
  scale2 = ln_scale.reshape(1, D_EMB)
  bias2 = ln_bias.reshape(1, D_EMB)
  db2 = dense_bias.reshape(1, D_MODEL)

  n_blocks = n_tok // TC_BLOCK
  pos_blocks = seq // TC_BLOCK

  out = pl.pallas_call(
      _tc_body,
      grid=(n_blocks,),
      in_specs=[
          pl.BlockSpec((TC_BLOCK, D_EMB), lambda i: (i, 0)),
          pl.BlockSpec((TC_BLOCK, D_EMB), lambda i: (i % pos_blocks, 0)),
          pl.BlockSpec((TC_BLOCK, 1), lambda i: (i, 0)),
          pl.BlockSpec((2, D_EMB), lambda i: (0, 0)),
          pl.BlockSpec((1, D_EMB), lambda i: (0, 0)),
          pl.BlockSpec((1, D_EMB), lambda i: (0, 0)),
          pl.BlockSpec((D_EMB, D_MODEL), lambda i: (0, 0)),
          pl.BlockSpec((1, D_MODEL), lambda i: (0, 0)),
      ],
      out_specs=pl.BlockSpec((TC_BLOCK, D_MODEL), lambda i: (i, 0)),
      out_shape=jax.ShapeDtypeStruct((n_tok, D_MODEL), jnp.float32),
  )(gathered, pos2, tids, type_emb, scale2, bias2, dense_kernel, db2)

  return out.reshape(batch, seq, D_MODEL)


# speedup vs baseline: 1.4020x; 1.1034x over previous
import functools

import jax
import jax.numpy as jnp
from jax import lax
from jax.experimental import pallas as pl
from jax.experimental.pallas import tpu as pltpu
from jax.experimental.pallas import tpu_sc as plsc

VOCAB = 100000
D_EMB = 128
MAX_SEQ = 2048
D_MODEL = 1024
LN_EPS = 1e-12

N_TOK = 8192
NW = 32
TOK_PER_TILE = N_TOK // NW
ROWS_PER_GATHER = 128

TC_BLOCK = 1024


@functools.cache
def _make_sc_gather():
  mesh = plsc.VectorSubcoreMesh(core_axis_name="c", subcore_axis_name="s")

  @functools.partial(
      pl.kernel,
      mesh=mesh,
      out_type=jax.ShapeDtypeStruct((N_TOK, D_EMB), jnp.float32),
      scratch_types=[
          pltpu.VMEM((2, ROWS_PER_GATHER), jnp.int32),
          pltpu.VMEM((TOK_PER_TILE, D_EMB), jnp.float32),
          pltpu.SemaphoreType.DMA,
      ],
  )
  def gather_kernel(ids_hbm, table_hbm, out_hbm, idx_v, rows_v, sem):
    c = lax.axis_index("c")
    s = lax.axis_index("s")
    wid = s * 2 + c
    pltpu.sync_copy(ids_hbm.at[pl.ds(wid * 2, 2)], idx_v)
    cp0 = pltpu.async_copy(
        table_hbm.at[idx_v.at[0]], rows_v.at[pl.ds(0, ROWS_PER_GATHER)], sem)
    cp1 = pltpu.async_copy(
        table_hbm.at[idx_v.at[1]],
        rows_v.at[pl.ds(ROWS_PER_GATHER, ROWS_PER_GATHER)], sem)
    cp0.wait()
    cp1.wait()
    pltpu.sync_copy(rows_v, out_hbm.at[pl.ds(wid * TOK_PER_TILE, TOK_PER_TILE)])

  return gather_kernel


def _tc_body(gath_ref, pos_ref, tid_ref, temb_ref, scale_ref, bias_ref,
             dk_ref, db_ref, out_ref):
  x = gath_ref[...] + pos_ref[...]
  t = tid_ref[...].astype(jnp.float32)
  te0 = temb_ref[0:1, :]
  te1 = temb_ref[1:2, :]
  x = x + te0 + t * (te1 - te0)
  mean = jnp.mean(x, axis=1, keepdims=True)
  xc = x - mean
  var = jnp.mean(xc * xc, axis=1, keepdims=True)
  y = xc * lax.rsqrt(var + LN_EPS)
  y = y * scale_ref[...] + bias_ref[...]
  out_ref[...] = (
      jnp.dot(y, dk_ref[...], preferred_element_type=jnp.float32)
      + db_ref[...])


def kernel(input_ids, type_ids, word_emb, pos_emb, type_emb, ln_scale,
           ln_bias, dense_kernel, dense_bias):
  batch, seq = input_ids.shape
  n_tok = batch * seq

  ids2 = input_ids.reshape(NW * 2, ROWS_PER_GATHER).astype(jnp.int32)
  gathered = _make_sc_gather()(ids2, word_emb)

  pos2 = pos_emb.reshape(MAX_SEQ, D_EMB)[:seq]
  tids = type_ids.reshape(n_tok, 1).astype(jnp.int32)
  scale2 = ln_scale.reshape(1, D_EMB)
  bias2 = ln_bias.reshape(1, D_EMB)
  db2 = dense_bias.reshape(1, D_MODEL)

  n_blocks = n_tok // TC_BLOCK
  pos_blocks = seq // TC_BLOCK

  out = pl.pallas_call(
      _tc_body,
      grid=(n_blocks,),
      in_specs=[
          pl.BlockSpec((TC_BLOCK, D_EMB), lambda i: (i, 0)),
          pl.BlockSpec((TC_BLOCK, D_EMB), lambda i: (i % pos_blocks, 0)),
          pl.BlockSpec((TC_BLOCK, 1), lambda i: (i, 0)),
          pl.BlockSpec((2, D_EMB), lambda i: (0, 0)),
          pl.BlockSpec((1, D_EMB), lambda i: (0, 0)),
          pl.BlockSpec((1, D_EMB), lambda i: (0, 0)),
          pl.BlockSpec((D_EMB, D_MODEL), lambda i: (0, 0)),
          pl.BlockSpec((1, D_MODEL), lambda i: (0, 0)),
      ],
      out_specs=pl.BlockSpec((TC_BLOCK, D_MODEL), lambda i: (i, 0)),
      out_shape=jax.ShapeDtypeStruct((n_tok, D_MODEL), jnp.float32),
  )(gathered, pos2, tids, type_emb, scale2, bias2, dense_kernel, db2)

  return out.reshape(batch, seq, D_MODEL)


# speedup vs baseline: 1.4706x; 1.0490x over previous
import functools

import jax
import jax.numpy as jnp
from jax import lax
from jax.experimental import pallas as pl
from jax.experimental.pallas import tpu as pltpu
from jax.experimental.pallas import tpu_sc as plsc

VOCAB = 100000
D_EMB = 128
MAX_SEQ = 2048
D_MODEL = 1024
LN_EPS = 1e-12

N_TOK = 8192
NW = 32
TOK_PER_TILE = N_TOK // NW
ROWS_PER_GATHER = 128

TC_BLOCK = 2048


@functools.cache
def _make_sc_gather():
  mesh = plsc.VectorSubcoreMesh(core_axis_name="c", subcore_axis_name="s")

  @functools.partial(
      pl.kernel,
      mesh=mesh,
      out_type=jax.ShapeDtypeStruct((N_TOK, D_EMB), jnp.float32),
      scratch_types=[
          pltpu.VMEM((2, ROWS_PER_GATHER), jnp.int32),
          pltpu.VMEM((TOK_PER_TILE, D_EMB), jnp.float32),
          pltpu.SemaphoreType.DMA,
      ],
  )
  def gather_kernel(ids_hbm, table_hbm, out_hbm, idx_v, rows_v, sem):
    c = lax.axis_index("c")
    s = lax.axis_index("s")
    wid = s * 2 + c
    pltpu.sync_copy(ids_hbm.at[pl.ds(wid * 2, 2)], idx_v)
    cp0 = pltpu.async_copy(
        table_hbm.at[idx_v.at[0]], rows_v.at[pl.ds(0, ROWS_PER_GATHER)], sem)
    cp1 = pltpu.async_copy(
        table_hbm.at[idx_v.at[1]],
        rows_v.at[pl.ds(ROWS_PER_GATHER, ROWS_PER_GATHER)], sem)
    cp0.wait()
    cp1.wait()
    pltpu.sync_copy(rows_v, out_hbm.at[pl.ds(wid * TOK_PER_TILE, TOK_PER_TILE)])

  return gather_kernel


def _tc_body(gath_ref, pos_ref, tid_ref, temb_ref, scale_ref, bias_ref,
             dk_ref, db_ref, out_ref):
  x = gath_ref[...] + pos_ref[...]
  t = tid_ref[...].astype(jnp.float32)
  te0 = temb_ref[0:1, :]
  te1 = temb_ref[1:2, :]
  x = x + te0 + t * (te1 - te0)
  mean = jnp.mean(x, axis=1, keepdims=True)
  xc = x - mean
  var = jnp.mean(xc * xc, axis=1, keepdims=True)
  y = xc * lax.rsqrt(var + LN_EPS)
  y = y * scale_ref[...] + bias_ref[...]
  out_ref[...] = (
      jnp.dot(y, dk_ref[...], preferred_element_type=jnp.float32)
      + db_ref[...])


def kernel(input_ids, type_ids, word_emb, pos_emb, type_emb, ln_scale,
           ln_bias, dense_kernel, dense_bias):
  batch, seq = input_ids.shape
  n_tok = batch * seq

  ids2 = input_ids.reshape(NW * 2, ROWS_PER_GATHER).astype(jnp.int32)
  gathered = _make_sc_gather()(ids2, word_emb)

  pos2 = pos_emb.reshape(MAX_SEQ, D_EMB)[:seq]
  tids = type_ids.reshape(n_tok, 1).astype(jnp.int32)
  scale2 = ln_scale.reshape(1, D_EMB)
  bias2 = ln_bias.reshape(1, D_EMB)
  db2 = dense_bias.reshape(1, D_MODEL)

  n_blocks = n_tok // TC_BLOCK
  pos_blocks = seq // TC_BLOCK

  out = pl.pallas_call(
      _tc_body,
      grid=(n_blocks,),
      in_specs=[
          pl.BlockSpec((TC_BLOCK, D_EMB), lambda i: (i, 0)),
          pl.BlockSpec((TC_BLOCK, D_EMB), lambda i: (i % pos_blocks, 0)),
          pl.BlockSpec((TC_BLOCK, 1), lambda i: (i, 0)),
          pl.BlockSpec((2, D_EMB), lambda i: (0, 0)),
          pl.BlockSpec((1, D_EMB), lambda i: (0, 0)),
          pl.BlockSpec((1, D_EMB), lambda i: (0, 0)),
          pl.BlockSpec((D_EMB, D_MODEL), lambda i: (0, 0)),
          pl.BlockSpec((1, D_MODEL), lambda i: (0, 0)),
      ],
      out_specs=pl.BlockSpec((TC_BLOCK, D_MODEL), lambda i: (i, 0)),
      out_shape=jax.ShapeDtypeStruct((n_tok, D_MODEL), jnp.float32),
  )(gathered, pos2, tids, type_emb, scale2, bias2, dense_kernel, db2)

  return out.reshape(batch, seq, D_MODEL)
